# Initial kernel scaffold; baseline (speedup 1.0000x reference)
import jax
import jax.numpy as jnp
from jax.experimental import pallas as pl


def kernel(x, edge_index, W1, b1, W2, b2):
    raise NotImplementedError("write your pallas kernel here")


# speedup vs baseline: 29.0447x; 29.0447x over previous
import functools

import jax
import jax.numpy as jnp
from jax import lax
from jax.experimental import pallas as pl
from jax.experimental.pallas import tpu as pltpu
from jax.experimental.pallas import tpu_sc as plsc

N = 10000
NP = 10240
E = 320000
NC = 2
NS = 16
NW = NC * NS
EPW = E // NW
CHUNK = 80
NCHUNK = EPW // CHUNK
ROWS_PT = NP // NS

_MESH = plsc.VectorSubcoreMesh(core_axis_name="c", subcore_axis_name="s")


def _make_agg(D):
  @functools.partial(
      pl.kernel,
      out_type=jax.ShapeDtypeStruct((NC, NP, D), jnp.float32),
      mesh=_MESH,
      compiler_params=pltpu.CompilerParams(use_tc_tiling_on_sc=False),
      scratch_types=[
          pltpu.VMEM((NCHUNK, CHUNK), jnp.int32),
          pltpu.VMEM((NCHUNK, CHUNK), jnp.int32),
          pltpu.VMEM((CHUNK, D), jnp.float32),
          pltpu.VMEM((CHUNK, D), jnp.float32),
          pltpu.VMEM_SHARED((NP, D), jnp.float32),
          pltpu.SemaphoreType.DMA,
          pltpu.SemaphoreType.DMA,
      ],
  )
  def agg(src_hbm, dst_hbm, g_hbm, zeros_hbm, out_hbm,
          idx_s, idx_d, rows0, rows1, acc, sem0, sem1):
    cid = lax.axis_index("c")
    sid = lax.axis_index("s")
    wid = cid * NS + sid
    r0 = sid * ROWS_PT

    @pl.when(cid == 0)
    def _():
      pltpu.sync_copy(g_hbm.at[pl.ds(r0, ROWS_PT)], acc.at[pl.ds(r0, ROWS_PT)])

    @pl.when(cid != 0)
    def _():
      pltpu.sync_copy(zeros_hbm.at[pl.ds(r0, ROWS_PT)],
                      acc.at[pl.ds(r0, ROWS_PT)])

    pltpu.sync_copy(src_hbm.at[wid], idx_s)
    pltpu.sync_copy(dst_hbm.at[wid], idx_d)
    plsc.subcore_barrier()

    @pl.loop(0, NCHUNK - 1, step=2)
    def _(j):
      ga = pltpu.async_copy(g_hbm.at[idx_s.at[j]], rows0, sem0)
      gb = pltpu.async_copy(g_hbm.at[idx_s.at[j + 1]], rows1, sem1)
      ga.wait()
      pltpu.sync_copy(rows0, acc.at[idx_d.at[j]], add=True)
      gb.wait()
      pltpu.sync_copy(rows1, acc.at[idx_d.at[j + 1]], add=True)

    j = NCHUNK - 1
    pltpu.async_copy(g_hbm.at[idx_s.at[j]], rows0, sem0).wait()
    pltpu.sync_copy(rows0, acc.at[idx_d.at[j]], add=True)

    plsc.subcore_barrier()
    pltpu.sync_copy(acc.at[pl.ds(r0, ROWS_PT)],
                    out_hbm.at[cid, pl.ds(r0, ROWS_PT)])

  return agg


_agg32 = _make_agg(32)
_agg128 = _make_agg(128)

DEG_W = 8


@functools.partial(
    pl.kernel,
    out_type=jax.ShapeDtypeStruct((NC, NP, DEG_W), jnp.float32),
    mesh=_MESH,
    compiler_params=pltpu.CompilerParams(use_tc_tiling_on_sc=False),
    scratch_types=[
        pltpu.VMEM((NCHUNK, CHUNK), jnp.int32),
        pltpu.VMEM((CHUNK, DEG_W), jnp.float32),
        pltpu.VMEM_SHARED((NP, DEG_W), jnp.float32),
    ],
)
def _deg(dst_hbm, ones_n_hbm, zeros_n_hbm, ones_c_hbm, out_hbm,
         idx_d, ones_v, acc):
  cid = lax.axis_index("c")
  sid = lax.axis_index("s")
  wid = cid * NS + sid
  r0 = sid * ROWS_PT

  @pl.when(cid == 0)
  def _():
    pltpu.sync_copy(ones_n_hbm.at[pl.ds(r0, ROWS_PT)],
                    acc.at[pl.ds(r0, ROWS_PT)])

  @pl.when(cid != 0)
  def _():
    pltpu.sync_copy(zeros_n_hbm.at[pl.ds(r0, ROWS_PT)],
                    acc.at[pl.ds(r0, ROWS_PT)])

  pltpu.sync_copy(dst_hbm.at[wid], idx_d)
  pltpu.sync_copy(ones_c_hbm, ones_v)
  plsc.subcore_barrier()

  @pl.loop(0, NCHUNK)
  def _(j):
    pltpu.sync_copy(ones_v, acc.at[idx_d.at[j]], add=True)

  plsc.subcore_barrier()
  pltpu.sync_copy(acc.at[pl.ds(r0, ROWS_PT)],
                  out_hbm.at[cid, pl.ds(r0, ROWS_PT)])


def _tc_first(deg_ref, x_ref, w1_ref, g1_ref, dinv_ref):
  deg = deg_ref[0, :N, 0:1] + deg_ref[1, :N, 0:1]
  dinv = lax.rsqrt(deg)
  h1 = jnp.dot(x_ref[...], w1_ref[...], preferred_element_type=jnp.float32)
  g1_ref[:N] = h1 * dinv
  g1_ref[N:] = jnp.zeros((NP - N, 32), jnp.float32)
  dinv_ref[...] = dinv


def _tc_mid(s1_ref, dinv_ref, b1_ref, w2_ref, g2_ref):
  dinv = dinv_ref[...]
  t = (s1_ref[0, :N] + s1_ref[1, :N]) * dinv + b1_ref[...]
  h = jnp.maximum(t, 0.0)
  h2 = jnp.dot(h, w2_ref[...], preferred_element_type=jnp.float32)
  g2_ref[:N] = h2 * dinv
  g2_ref[N:] = jnp.zeros((NP - N, 128), jnp.float32)


def _tc_last(s2_ref, dinv_ref, b2_ref, out_ref):
  out_ref[...] = (s2_ref[0, :N] + s2_ref[1, :N]) * dinv_ref[...] + b2_ref[...]


def kernel(x, edge_index, W1, b1, W2, b2):
  src = edge_index[0].astype(jnp.int32).reshape(NW, NCHUNK, CHUNK)
  dst = edge_index[1].astype(jnp.int32).reshape(NW, NCHUNK, CHUNK)

  zeros32 = jnp.zeros((NP, 32), jnp.float32)
  zeros128 = jnp.zeros((NP, 128), jnp.float32)
  ones_n = jnp.ones((NP, DEG_W), jnp.float32)
  zeros_n = jnp.zeros((NP, DEG_W), jnp.float32)
  ones_c = jnp.ones((CHUNK, DEG_W), jnp.float32)

  deg_p = _deg(dst, ones_n, zeros_n, ones_c)

  g1, dinv = pl.pallas_call(
      _tc_first,
      out_shape=[
          jax.ShapeDtypeStruct((NP, 32), jnp.float32),
          jax.ShapeDtypeStruct((N, 1), jnp.float32),
      ],
  )(deg_p, x, W1)

  s1 = _agg32(src, dst, g1, zeros32)

  g2 = pl.pallas_call(
      _tc_mid,
      out_shape=jax.ShapeDtypeStruct((NP, 128), jnp.float32),
  )(s1, dinv, b1.reshape(1, 32), W2)

  s2 = _agg128(src, dst, g2, zeros128)

  return pl.pallas_call(
      _tc_last,
      out_shape=jax.ShapeDtypeStruct((N, 128), jnp.float32),
  )(s2, dinv, b2.reshape(1, 128))


# speedup vs baseline: 37.6953x; 1.2978x over previous
import functools

import jax
import jax.numpy as jnp
from jax import lax
from jax.experimental import pallas as pl
from jax.experimental.pallas import tpu as pltpu
from jax.experimental.pallas import tpu_sc as plsc

N = 10000
NP = 10240
E = 320000
NC = 2
NS = 16
NW = NC * NS
EPW = E // NW
CHUNK = 80
NCHUNK = EPW // CHUNK
ROWS_PT = NP // NS

_MESH = plsc.VectorSubcoreMesh(core_axis_name="c", subcore_axis_name="s")


def _make_agg(D):
  @functools.partial(
      pl.kernel,
      out_type=jax.ShapeDtypeStruct((NC, NP, D), jnp.float32),
      mesh=_MESH,
      compiler_params=pltpu.CompilerParams(use_tc_tiling_on_sc=False),
      scratch_types=[
          pltpu.VMEM((NCHUNK, CHUNK), jnp.int32),
          pltpu.VMEM((NCHUNK, CHUNK), jnp.int32),
          pltpu.VMEM((CHUNK, D), jnp.float32),
          pltpu.VMEM((CHUNK, D), jnp.float32),
          pltpu.VMEM_SHARED((NP, D), jnp.float32),
          pltpu.SemaphoreType.DMA,
          pltpu.SemaphoreType.DMA,
      ],
  )
  def agg(src_hbm, dst_hbm, g_hbm, zeros_hbm, out_hbm,
          idx_s, idx_d, rows0, rows1, acc, sem0, sem1):
    cid = lax.axis_index("c")
    sid = lax.axis_index("s")
    wid = cid * NS + sid
    r0 = sid * ROWS_PT

    @pl.when(cid == 0)
    def _():
      pltpu.sync_copy(g_hbm.at[pl.ds(r0, ROWS_PT)], acc.at[pl.ds(r0, ROWS_PT)])

    @pl.when(cid != 0)
    def _():
      pltpu.sync_copy(zeros_hbm.at[pl.ds(r0, ROWS_PT)],
                      acc.at[pl.ds(r0, ROWS_PT)])

    pltpu.sync_copy(src_hbm.at[wid], idx_s)
    pltpu.sync_copy(dst_hbm.at[wid], idx_d)
    plsc.subcore_barrier()

    @pl.loop(0, NCHUNK - 1, step=2)
    def _(j):
      ga = pltpu.async_copy(g_hbm.at[idx_s.at[j]], rows0, sem0)
      gb = pltpu.async_copy(g_hbm.at[idx_s.at[j + 1]], rows1, sem1)
      ga.wait()
      pltpu.sync_copy(rows0, acc.at[idx_d.at[j]], add=True)
      gb.wait()
      pltpu.sync_copy(rows1, acc.at[idx_d.at[j + 1]], add=True)

    j = NCHUNK - 1
    pltpu.async_copy(g_hbm.at[idx_s.at[j]], rows0, sem0).wait()
    pltpu.sync_copy(rows0, acc.at[idx_d.at[j]], add=True)

    plsc.subcore_barrier()
    pltpu.sync_copy(acc.at[pl.ds(r0, ROWS_PT)],
                    out_hbm.at[cid, pl.ds(r0, ROWS_PT)])

  return agg


_agg32 = _make_agg(32)

DEG_W = 8


@functools.partial(
    pl.kernel,
    out_type=jax.ShapeDtypeStruct((NC, NP, DEG_W), jnp.float32),
    mesh=_MESH,
    compiler_params=pltpu.CompilerParams(use_tc_tiling_on_sc=False),
    scratch_types=[
        pltpu.VMEM((NCHUNK, CHUNK), jnp.int32),
        pltpu.VMEM((CHUNK, DEG_W), jnp.float32),
        pltpu.VMEM_SHARED((NP, DEG_W), jnp.float32),
    ],
)
def _deg(dst_hbm, ones_n_hbm, zeros_n_hbm, ones_c_hbm, out_hbm,
         idx_d, ones_v, acc):
  cid = lax.axis_index("c")
  sid = lax.axis_index("s")
  wid = cid * NS + sid
  r0 = sid * ROWS_PT

  @pl.when(cid == 0)
  def _():
    pltpu.sync_copy(ones_n_hbm.at[pl.ds(r0, ROWS_PT)],
                    acc.at[pl.ds(r0, ROWS_PT)])

  @pl.when(cid != 0)
  def _():
    pltpu.sync_copy(zeros_n_hbm.at[pl.ds(r0, ROWS_PT)],
                    acc.at[pl.ds(r0, ROWS_PT)])

  pltpu.sync_copy(dst_hbm.at[wid], idx_d)
  pltpu.sync_copy(ones_c_hbm, ones_v)
  plsc.subcore_barrier()

  @pl.loop(0, NCHUNK)
  def _(j):
    pltpu.sync_copy(ones_v, acc.at[idx_d.at[j]], add=True)

  plsc.subcore_barrier()
  pltpu.sync_copy(acc.at[pl.ds(r0, ROWS_PT)],
                  out_hbm.at[cid, pl.ds(r0, ROWS_PT)])


def _tc_first(deg_ref, x_ref, w1_ref, g1_ref, dinv_ref):
  deg = deg_ref[0, :N, 0:1] + deg_ref[1, :N, 0:1]
  dinv = lax.rsqrt(deg)
  h1 = jnp.dot(x_ref[...], w1_ref[...], preferred_element_type=jnp.float32)
  g1_ref[:N] = h1 * dinv
  g1_ref[N:] = jnp.zeros((NP - N, 32), jnp.float32)
  dinv_ref[...] = dinv


def _tc_mid(s1_ref, dinv_ref, b1_ref, g2_ref):
  dinv = dinv_ref[...]
  t = (s1_ref[0, :N] + s1_ref[1, :N]) * dinv + b1_ref[...]
  h = jnp.maximum(t, 0.0)
  g2_ref[:N] = h * dinv
  g2_ref[N:] = jnp.zeros((NP - N, 32), jnp.float32)


def _tc_last(s2_ref, dinv_ref, w2_ref, b2_ref, out_ref):
  t = (s2_ref[0, :N] + s2_ref[1, :N]) * dinv_ref[...]
  out_ref[...] = (
      jnp.dot(t, w2_ref[...], preferred_element_type=jnp.float32)
      + b2_ref[...]
  )


def kernel(x, edge_index, W1, b1, W2, b2):
  src = edge_index[0].astype(jnp.int32).reshape(NW, NCHUNK, CHUNK)
  dst = edge_index[1].astype(jnp.int32).reshape(NW, NCHUNK, CHUNK)

  zeros32 = jnp.zeros((NP, 32), jnp.float32)
  ones_n = jnp.ones((NP, DEG_W), jnp.float32)
  zeros_n = jnp.zeros((NP, DEG_W), jnp.float32)
  ones_c = jnp.ones((CHUNK, DEG_W), jnp.float32)

  deg_p = _deg(dst, ones_n, zeros_n, ones_c)

  g1, dinv = pl.pallas_call(
      _tc_first,
      out_shape=[
          jax.ShapeDtypeStruct((NP, 32), jnp.float32),
          jax.ShapeDtypeStruct((N, 1), jnp.float32),
      ],
  )(deg_p, x, W1)

  s1 = _agg32(src, dst, g1, zeros32)

  g2 = pl.pallas_call(
      _tc_mid,
      out_shape=jax.ShapeDtypeStruct((NP, 32), jnp.float32),
  )(s1, dinv, b1.reshape(1, 32))

  s2 = _agg32(src, dst, g2, zeros32)

  return pl.pallas_call(
      _tc_last,
      out_shape=jax.ShapeDtypeStruct((N, 128), jnp.float32),
  )(s2, dinv, W2, b2.reshape(1, 128))
